# baseline (speedup 1.0000x reference)
import jax
import jax.numpy as jnp
from jax import lax
from jax.experimental import pallas as pl
from jax.experimental.pallas import tpu as pltpu
from jax.experimental.pallas import tpu_sc as plsc

N_VOCAB_ = 49408
N_EMBD_ = 768
N_TOKEN_ = 77
BATCH_ = 1024

NC = 2
NS = 16
LANES = 16
NW = NC * NS

B_PER_W = BATCH_ // NW
NB_CH = 8
TB = 8
NBAND = 10
CH_PER_BAND = B_PER_W // NB_CH
NCH = NBAND * CH_PER_BAND
ROWS = NB_CH * TB
CW = 5
D_SLICES = N_EMBD_ // LANES
LAST_MAIN = NCH - CH_PER_BAND - 1


def _body(idx_hbm, tab_hbm, pos_hbm, out_hbm,
          idx_v, pos_v, scr, bufW,
          sg0, sg1, sw0, sw1, swW):
    wid = lax.axis_index("s") * NC + lax.axis_index("c")
    base_batch = wid * B_PER_W

    wins = (scr.at[pl.ds(0, ROWS)], scr.at[pl.ds(ROWS, ROWS)])
    sgs = (sg0, sg1)
    sws = (sw0, sw1)

    pltpu.sync_copy(idx_hbm.at[wid], idx_v)

    def start_gather(q, w, sem):
        pltpu.async_copy(tab_hbm.at[idx_v.at[q]], wins[w], sem)

    def step(q, w):
        win = wins[w]
        k = q // CH_PER_BAND
        c = lax.rem(q, CH_PER_BAND)
        bb = base_batch + c * NB_CH
        t0 = pl.multiple_of(k * TB, TB)

        @pl.when(c == 0)
        def _():
            pltpu.sync_copy(pos_hbm.at[pl.ds(t0, TB)], pos_v)

        pltpu.make_async_copy(tab_hbm.at[idx_v.at[q]], win, sgs[w]).wait()

        @pl.when(q + 1 < NCH)
        def _():
            w2 = 1 - w

            @pl.when(jnp.logical_and(q >= 1, q <= LAST_MAIN + 1))
            def _():
                for i in range(NB_CH):
                    pltpu.make_async_copy(
                        wins[w2].at[pl.ds(i * TB, TB)],
                        out_hbm.at[base_batch, pl.ds(0, TB)],
                        sws[w2]).wait()

            start_gather(q + 1, w2, sgs[w2])

        @pl.when(k < NBAND - 1)
        def _():
            def add_body(j, _):
                for u in range(2):
                    sl = pl.ds((2 * j + u) * LANES, LANES)
                    for r8 in range(TB):
                        v = pos_v[r8, sl]
                        for i in range(NB_CH):
                            plsc.addupdate(win.at[i * TB + r8, sl], v)
                return 0

            lax.fori_loop(0, D_SLICES // 2, add_body, 0)
            for i in range(NB_CH):
                pltpu.async_copy(win.at[pl.ds(i * TB, TB)],
                                 out_hbm.at[bb + i, pl.ds(t0, TB)],
                                 sws[w])

        @pl.when(k == NBAND - 1)
        def _():
            for i in range(NB_CH):
                if i == 0:
                    @pl.when(q > NCH - CH_PER_BAND)
                    def _():
                        pltpu.make_async_copy(
                            bufW,
                            out_hbm.at[base_batch, pl.ds(N_TOKEN_ - CW, CW)],
                            swW).wait()
                else:
                    pltpu.make_async_copy(
                        bufW,
                        out_hbm.at[base_batch, pl.ds(N_TOKEN_ - CW, CW)],
                        swW).wait()

                def tail_body(j, _, i=i):
                    for u in range(2):
                        sl = pl.ds((2 * j + u) * LANES, LANES)
                        for r in range(CW):
                            bufW[r, sl] = win[i * TB + r, sl] + pos_v[r, sl]
                    return 0

                lax.fori_loop(0, D_SLICES // 2, tail_body, 0)
                pltpu.async_copy(bufW,
                                 out_hbm.at[bb + i, pl.ds(N_TOKEN_ - CW, CW)],
                                 swW)


    def pair_body(p, _):
        step(p * 2, 0)
        step(p * 2 + 1, 1)
        return 0

    start_gather(0, 0, sg0)
    lax.fori_loop(0, NCH // 2, pair_body, 0)
    pltpu.make_async_copy(bufW,
                          out_hbm.at[base_batch, pl.ds(N_TOKEN_ - CW, CW)],
                          swW).wait()


@jax.jit
def kernel(tokens, embedding_token, embedding_posicao):
    mesh = plsc.VectorSubcoreMesh(core_axis_name="c", subcore_axis_name="s")
    tok = tokens.astype(jnp.int32)
    tok_pad = jnp.concatenate([tok, tok[:, N_TOKEN_ - 3:]], axis=1)
    idx = tok_pad.reshape(NW, CH_PER_BAND, NB_CH, NBAND, TB)
    idx = idx.transpose(0, 3, 1, 2, 4).reshape(NW, NCH, ROWS)
    pos_pad = jnp.concatenate(
        [embedding_posicao,
         jnp.zeros((NBAND * TB - N_TOKEN_, N_EMBD_), jnp.float32)], axis=0)
    out = pl.kernel(
        _body,
        out_type=jax.ShapeDtypeStruct((BATCH_, N_TOKEN_, N_EMBD_), jnp.float32),
        mesh=mesh,
        scratch_types=[
            pltpu.VMEM((NCH, ROWS), jnp.int32),
            pltpu.VMEM((TB, N_EMBD_), jnp.float32),
            pltpu.VMEM((2 * ROWS, N_EMBD_), jnp.float32),
            pltpu.VMEM((CW, N_EMBD_), jnp.float32),
            pltpu.SemaphoreType.DMA,
            pltpu.SemaphoreType.DMA,
            pltpu.SemaphoreType.DMA,
            pltpu.SemaphoreType.DMA,
            pltpu.SemaphoreType.DMA,
        ],
    )(idx, embedding_token, pos_pad)
    return out
